# baseline (speedup 1.0000x reference)
import functools

import jax
import jax.numpy as jnp
from jax import lax
from jax.experimental import pallas as pl
from jax.experimental.pallas import tpu as pltpu
from jax.experimental.pallas import tpu_sc as plsc

B = 16384
D = 16
_NC = 2
_NS = 16
_NW = _NC * _NS
_BPW = B // _NW
_G = 16
_NGRP = _BPW // _G


def _gather_table(tab_t, ids_v, win, out_v, sem):
    iota = lax.iota(jnp.int32, 16)

    def group(g, _):
        v = ids_v[pl.ds(g * _G, _G)]
        for u in range(_G):
            colb = lax.mul(lax.shift_right_logical(v[u], 7), 128)
            pltpu.async_copy(tab_t.at[:, pl.ds(colb, 128)], win.at[u], sem)
        pltpu.make_async_copy(tab_t.at[:, pl.ds(0, _G * 128)], win, sem).wait()
        for u in range(_G):
            col = v[u] & 127
            vals = plsc.load_gather(win.at[u], [iota, jnp.full((16,), col, jnp.int32)])
            plsc.store_scatter(
                out_v, [iota, jnp.full((16,), g * _G + u, jnp.int32)], vals)
        return 0

    lax.fori_loop(0, _NGRP, group, 0)


def _gather_body(uid_hbm, iid_hbm, utab_t, itab_t, x_hbm,
                 uids_v, iids_v, win, uout_v, iout_v, sem):
    wid = lax.axis_index("s") * _NC + lax.axis_index("c")
    base = wid * _BPW
    pltpu.sync_copy(uid_hbm.at[pl.ds(base, _BPW)], uids_v)
    pltpu.sync_copy(iid_hbm.at[pl.ds(base, _BPW)], iids_v)
    _gather_table(utab_t, uids_v, win, uout_v, sem)
    _gather_table(itab_t, iids_v, win, iout_v, sem)
    pltpu.sync_copy(uout_v, x_hbm.at[pl.ds(0, D), pl.ds(base, _BPW)])
    pltpu.sync_copy(iout_v, x_hbm.at[pl.ds(D, D), pl.ds(base, _BPW)])


_gather = functools.partial(
    pl.kernel,
    out_type=jax.ShapeDtypeStruct((2 * D, B), jnp.float32),
    mesh=plsc.VectorSubcoreMesh(core_axis_name="c", subcore_axis_name="s"),
    scratch_types=[
        pltpu.VMEM((_BPW,), jnp.int32),
        pltpu.VMEM((_BPW,), jnp.int32),
        pltpu.VMEM((_G, D, 128), jnp.float32),
        pltpu.VMEM((D, _BPW), jnp.float32),
        pltpu.VMEM((D, _BPW), jnp.float32),
        pltpu.SemaphoreType.DMA,
    ],
    compiler_params=pltpu.CompilerParams(needs_layout_passes=False),
)(_gather_body)


def _bn_lrelu(h, g, be):
    mu = jnp.mean(h, axis=1, keepdims=True)
    c = h - mu
    var = jnp.mean(c * c, axis=1, keepdims=True)
    h = c * lax.rsqrt(var + 1e-5) * g + be
    return jnp.where(h >= 0, h, 0.2 * h)


def _dense_body(x_ref, gmf_w_ref, gmf_b_ref,
                w1_ref, b1_ref, g1_ref, be1_ref,
                w2_ref, b2_ref, g2_ref, be2_ref,
                w3_ref, b3_ref, g3_ref, be3_ref,
                wo_ref, bo_ref, wfu_ref, wfi_ref, bf_ref, out_ref):
    x = x_ref[:]
    hp = lax.Precision.HIGHEST
    prod = x[0:D, :] * x[D:2 * D, :]
    gmf = jnp.dot(gmf_w_ref[:], prod, precision=hp,
                  preferred_element_type=jnp.float32) + gmf_b_ref[:]
    h = jnp.dot(w1_ref[:], x, precision=hp,
                preferred_element_type=jnp.float32) + b1_ref[:]
    h = _bn_lrelu(h, g1_ref[:], be1_ref[:])
    h = jnp.dot(w2_ref[:], h, precision=hp,
                preferred_element_type=jnp.float32) + b2_ref[:]
    h = _bn_lrelu(h, g2_ref[:], be2_ref[:])
    h = jnp.dot(w3_ref[:], h, precision=hp,
                preferred_element_type=jnp.float32) + b3_ref[:]
    h = _bn_lrelu(h, g3_ref[:], be3_ref[:])
    mlp = jnp.dot(wo_ref[:], h, precision=hp,
                  preferred_element_type=jnp.float32) + bo_ref[:]
    out_ref[:] = gmf * wfu_ref[:] + mlp * wfi_ref[:] + bf_ref[:]


def kernel(user_ids, item_ids, user_table, item_table, gmf_w, gmf_b,
           w1, b1, g1, be1, w2, b2, g2, be2, w3, b3, g3, be3,
           wo, bo, wf, bf):
    x_t = _gather(user_ids.astype(jnp.int32), item_ids.astype(jnp.int32),
                  user_table.T, item_table.T)
    out = pl.pallas_call(
        _dense_body,
        out_shape=jax.ShapeDtypeStruct((1, B), jnp.float32),
    )(x_t,
      gmf_w, gmf_b.reshape(1, 1),
      w1, b1.reshape(-1, 1), g1.reshape(-1, 1), be1.reshape(-1, 1),
      w2, b2.reshape(-1, 1), g2.reshape(-1, 1), be2.reshape(-1, 1),
      w3, b3.reshape(-1, 1), g3.reshape(-1, 1), be3.reshape(-1, 1),
      wo, bo.reshape(1, 1),
      wf[:, 0:1], wf[:, 1:2], bf.reshape(1, 1))
    return out.reshape(B)


# speedup vs baseline: 1.1998x; 1.1998x over previous
import functools

import jax
import jax.numpy as jnp
from jax import lax
from jax.experimental import pallas as pl
from jax.experimental.pallas import tpu as pltpu
from jax.experimental.pallas import tpu_sc as plsc

B = 16384
D = 16
_NC = 2
_NS = 16
_NW = _NC * _NS
_BPW = B // _NW
_G = 16
_NGRP = _BPW // _G


def _gather_table(tab_t, ids_v, wina, winb, out_v, sema, semb):
    iota = lax.iota(jnp.int32, 16)

    def fire(g, win, sem):
        v = ids_v[pl.ds(g * _G, _G)]
        for u in range(_G):
            colb = lax.mul(lax.shift_right_logical(v[u], 7), 128)
            pltpu.async_copy(tab_t.at[:, pl.ds(colb, 128)], win.at[u], sem)

    def drain(win, sem):
        pltpu.make_async_copy(tab_t.at[:, pl.ds(0, _G * 128)], win, sem).wait()

    def extract(g, win):
        v = ids_v[pl.ds(g * _G, _G)]
        for u in range(_G):
            col = v[u] & 127
            vals = plsc.load_gather(
                win.at[u], [iota, jnp.full((16,), col, jnp.int32)])
            plsc.store_scatter(
                out_v, [iota, jnp.full((16,), g * _G + u, jnp.int32)], vals)

    fire(0, wina, sema)
    fire(1, winb, semb)

    def step(i, _):
        g = i * 2
        drain(wina, sema)
        extract(g, wina)
        fire(g + 2, wina, sema)
        drain(winb, semb)
        extract(g + 1, winb)
        fire(g + 3, winb, semb)
        return 0

    lax.fori_loop(0, _NGRP // 2 - 1, step, 0)
    drain(wina, sema)
    extract(_NGRP - 2, wina)
    drain(winb, semb)
    extract(_NGRP - 1, winb)


def _gather_body(uid_hbm, iid_hbm, utab_t, itab_t, x_hbm,
                 uids_v, iids_v, wina, winb, uout_v, iout_v, sema, semb):
    wid = lax.axis_index("s") * _NC + lax.axis_index("c")
    base = wid * _BPW
    pltpu.sync_copy(uid_hbm.at[pl.ds(base, _BPW)], uids_v)
    pltpu.sync_copy(iid_hbm.at[pl.ds(base, _BPW)], iids_v)
    _gather_table(utab_t, uids_v, wina, winb, uout_v, sema, semb)
    _gather_table(itab_t, iids_v, wina, winb, iout_v, sema, semb)
    pltpu.sync_copy(uout_v, x_hbm.at[pl.ds(0, D), pl.ds(base, _BPW)])
    pltpu.sync_copy(iout_v, x_hbm.at[pl.ds(D, D), pl.ds(base, _BPW)])


_gather = functools.partial(
    pl.kernel,
    out_type=jax.ShapeDtypeStruct((2 * D, B), jnp.float32),
    mesh=plsc.VectorSubcoreMesh(core_axis_name="c", subcore_axis_name="s"),
    scratch_types=[
        pltpu.VMEM((_BPW,), jnp.int32),
        pltpu.VMEM((_BPW,), jnp.int32),
        pltpu.VMEM((_G, D, 128), jnp.float32),
        pltpu.VMEM((_G, D, 128), jnp.float32),
        pltpu.VMEM((D, _BPW), jnp.float32),
        pltpu.VMEM((D, _BPW), jnp.float32),
        pltpu.SemaphoreType.DMA,
        pltpu.SemaphoreType.DMA,
    ],
    compiler_params=pltpu.CompilerParams(needs_layout_passes=False),
)(_gather_body)


def _bn_lrelu(h, g, be):
    mu = jnp.mean(h, axis=1, keepdims=True)
    c = h - mu
    var = jnp.mean(c * c, axis=1, keepdims=True)
    h = c * lax.rsqrt(var + 1e-5) * g + be
    return jnp.where(h >= 0, h, 0.2 * h)


def _dense_body(x_ref, gmf_w_ref, gmf_b_ref,
                w1_ref, b1_ref, g1_ref, be1_ref,
                w2_ref, b2_ref, g2_ref, be2_ref,
                w3_ref, b3_ref, g3_ref, be3_ref,
                wo_ref, bo_ref, wfu_ref, wfi_ref, bf_ref, out_ref):
    x = x_ref[:]
    hp = lax.Precision.HIGHEST
    prod = x[0:D, :] * x[D:2 * D, :]
    gmf = jnp.dot(gmf_w_ref[:], prod, precision=hp,
                  preferred_element_type=jnp.float32) + gmf_b_ref[:]
    h = jnp.dot(w1_ref[:], x, precision=hp,
                preferred_element_type=jnp.float32) + b1_ref[:]
    h = _bn_lrelu(h, g1_ref[:], be1_ref[:])
    h = jnp.dot(w2_ref[:], h, precision=hp,
                preferred_element_type=jnp.float32) + b2_ref[:]
    h = _bn_lrelu(h, g2_ref[:], be2_ref[:])
    h = jnp.dot(w3_ref[:], h, precision=hp,
                preferred_element_type=jnp.float32) + b3_ref[:]
    h = _bn_lrelu(h, g3_ref[:], be3_ref[:])
    mlp = jnp.dot(wo_ref[:], h, precision=hp,
                  preferred_element_type=jnp.float32) + bo_ref[:]
    out_ref[:] = gmf * wfu_ref[:] + mlp * wfi_ref[:] + bf_ref[:]


def kernel(user_ids, item_ids, user_table, item_table, gmf_w, gmf_b,
           w1, b1, g1, be1, w2, b2, g2, be2, w3, b3, g3, be3,
           wo, bo, wf, bf):
    x_t = _gather(user_ids.astype(jnp.int32), item_ids.astype(jnp.int32),
                  user_table.T, item_table.T)
    out = pl.pallas_call(
        _dense_body,
        out_shape=jax.ShapeDtypeStruct((1, B), jnp.float32),
    )(x_t,
      gmf_w, gmf_b.reshape(1, 1),
      w1, b1.reshape(-1, 1), g1.reshape(-1, 1), be1.reshape(-1, 1),
      w2, b2.reshape(-1, 1), g2.reshape(-1, 1), be2.reshape(-1, 1),
      w3, b3.reshape(-1, 1), g3.reshape(-1, 1), be3.reshape(-1, 1),
      wo, bo.reshape(1, 1),
      wf[:, 0:1], wf[:, 1:2], bf.reshape(1, 1))
    return out.reshape(B)
